# Initial kernel scaffold; baseline (speedup 1.0000x reference)
import jax
import jax.numpy as jnp
from jax.experimental import pallas as pl


def kernel(input_ids, word_table, pos_table):
    raise NotImplementedError("write your pallas kernel here")


# speedup vs baseline: 1.0772x; 1.0772x over previous
import functools

import jax
import jax.numpy as jnp
from jax import lax
from jax.experimental import pallas as pl
from jax.experimental.pallas import tpu as pltpu
from jax.experimental.pallas import tpu_sc as plsc

VOCAB = 100000
WORLD = 8
LOCAL_VOCAB = VOCAB // WORLD
HIDDEN = 1024
MAXSEQ = 2048
BATCH = 4
NTOK = BATCH * MAXSEQ

NC, NS, LANES = 2, 16, 16
NW = NC * NS
TPW = NTOK // NW
CHUNK = 32
NCHUNK = TPW // CHUNK

_mesh = plsc.VectorSubcoreMesh(core_axis_name="c", subcore_axis_name="s")


@functools.partial(
    pl.kernel,
    out_type=jax.ShapeDtypeStruct((NTOK, HIDDEN), jnp.float32),
    mesh=_mesh,
    scratch_types=[
        pltpu.VMEM((TPW,), jnp.int32),
        pltpu.VMEM((CHUNK, HIDDEN), jnp.float32),
        pltpu.VMEM((CHUNK, HIDDEN), jnp.float32),
        pltpu.SemaphoreType.DMA,
        pltpu.SemaphoreType.DMA,
    ],
)
def _embed(ids_hbm, word_hbm, pos_hbm, out_hbm, idx_v, wbuf, pbuf, gsem, psem):
    wid = lax.axis_index("s") * NC + lax.axis_index("c")
    base = wid * TPW
    pos_base = base % MAXSEQ

    pltpu.sync_copy(ids_hbm.at[pl.ds(base, TPW)], idx_v)
    for i in range(TPW // LANES):
        v = idx_v[pl.ds(i * LANES, LANES)]
        idx_v[pl.ds(i * LANES, LANES)] = jnp.where(v >= LOCAL_VOCAB, 0, v)

    def chunk_body(ci, _):
        row0 = ci * CHUNK
        g = pltpu.async_copy(word_hbm.at[idx_v.at[pl.ds(row0, CHUNK)]], wbuf, gsem)
        p = pltpu.async_copy(pos_hbm.at[pl.ds(pos_base + row0, CHUNK)], pbuf, psem)
        g.wait()
        p.wait()

        def row_body(r, _):
            def col_body(c2, _):
                sl = pl.ds(c2 * LANES, LANES)
                plsc.addupdate(wbuf.at[r, sl], pbuf[r, sl])
                return 0

            return lax.fori_loop(0, HIDDEN // LANES, col_body, 0)

        lax.fori_loop(0, CHUNK, row_body, 0)
        pltpu.sync_copy(wbuf, out_hbm.at[pl.ds(base + row0, CHUNK)])
        return 0

    lax.fori_loop(0, NCHUNK, chunk_body, 0)


def kernel(input_ids, word_table, pos_table):
    ids_flat = input_ids.reshape(NTOK)
    out = _embed(ids_flat, word_table, pos_table)
    return out.reshape(BATCH, MAXSEQ, HIDDEN)
